# baseline (speedup 1.0000x reference)
import functools

import jax
import jax.numpy as jnp
from jax import lax
from jax.experimental import pallas as pl
from jax.experimental.pallas import tpu as pltpu
from jax.experimental.pallas import tpu_sc as plsc

_B, _C, _N_IN, _N_OUT, _K = 4, 128, 65536, 16384, 8
_NC, _NS, _L = 2, 16, 16
_NW = _NC * _NS
_ROWS = _B * _C
_PAIRS_PW = _ROWS // (2 * _NW)
_SG = 8192
_NSG = _N_IN // _SG
_CH = 1024
_NCHUNK = _N_OUT // _CH


def _pool_body(x_hbm, nbr_hbm, out_hbm, packed_v, stage_v, idx_v, out_v,
               idx_sh, sem_nbr, sem_st0, sem_st1, sem_ix0, sem_ix1,
               sem_ot0, sem_ot1):
    cid = lax.axis_index("c")
    sid = lax.axis_index("s")
    wid = sid * _NC + cid
    sem_st = (sem_st0, sem_st1)
    sem_ix = (sem_ix0, sem_ix1)
    sem_ot = (sem_ot0, sem_ot1)

    @pl.when(sid == 0)
    def _stage():
        pltpu.async_copy(nbr_hbm, idx_sh, sem_nbr).wait()

    plsc.subcore_barrier()

    def pair_body(p, carry):
        ra = (wid * _PAIRS_PW + p) * 2
        rb = ra + 1

        pltpu.async_copy(idx_sh.at[:, pl.ds(0, _CH)], idx_v.at[0], sem_ix[0])

        _scope_pack = jax.named_scope("pool_pack")
        _scope_pack.__enter__()
        pltpu.async_copy(x_hbm.at[ra, pl.ds(0, _SG)], stage_v.at[0, 0],
                         sem_st[0])
        pltpu.async_copy(x_hbm.at[rb, pl.ds(0, _SG)], stage_v.at[0, 1],
                         sem_st[0])
        for s in range(_NSG):
            buf = s % 2
            if s + 1 < _NSG:
                nbuf = 1 - buf
                off = (s + 1) * _SG
                pltpu.async_copy(x_hbm.at[ra, pl.ds(off, _SG)],
                                 stage_v.at[nbuf, 0], sem_st[nbuf])
                pltpu.async_copy(x_hbm.at[rb, pl.ds(off, _SG)],
                                 stage_v.at[nbuf, 1], sem_st[nbuf])
            pltpu.make_async_copy(x_hbm.at[ra, pl.ds(0, _SG)],
                                  stage_v.at[buf, 0], sem_st[buf]).wait()
            pltpu.make_async_copy(x_hbm.at[rb, pl.ds(0, _SG)],
                                  stage_v.at[buf, 1], sem_st[buf]).wait()

            @plsc.parallel_loop(0, _SG // _L, unroll=4)
            def pack_body(j, _buf=buf, _s=s):
                base = j * _L
                a = stage_v[_buf, 0, pl.ds(base, _L)]
                b = stage_v[_buf, 1, pl.ds(base, _L)]
                pw = plsc.bitcast(
                    plsc.pack(a, b, format=plsc.PackFormat.INTERLEAVED),
                    jnp.int32)
                packed_v[pl.ds(_s * _SG + base, _L)] = pw

        _scope_pack.__exit__(None, None, None)

        _scope_gather = jax.named_scope("pool_gather")
        _scope_gather.__enter__()
        for cc in range(_NCHUNK):
            buf = cc % 2
            if cc + 1 < _NCHUNK:
                nbuf = 1 - buf
                pltpu.async_copy(idx_sh.at[:, pl.ds((cc + 1) * _CH, _CH)],
                                 idx_v.at[nbuf], sem_ix[nbuf])
            pltpu.make_async_copy(idx_sh.at[:, pl.ds(0, _CH)],
                                  idx_v.at[buf], sem_ix[buf]).wait()
            if cc >= 2:
                pltpu.make_async_copy(out_v.at[buf, 0],
                                      out_hbm.at[ra, pl.ds(0, _CH)],
                                      sem_ot[buf]).wait()
                pltpu.make_async_copy(out_v.at[buf, 1],
                                      out_hbm.at[rb, pl.ds(0, _CH)],
                                      sem_ot[buf]).wait()

            @plsc.parallel_loop(0, _CH // _L, unroll=2)
            def vec_body(j, _buf=buf):
                base = j * _L
                g = []
                for k in range(_K):
                    iv = idx_v[_buf, k, pl.ds(base, _L)]
                    gk = plsc.load_gather(packed_v, [iv])
                    g.append(plsc.bitcast(gk, jnp.bfloat16))
                m01 = jnp.maximum(g[0], g[1])
                m23 = jnp.maximum(g[2], g[3])
                m45 = jnp.maximum(g[4], g[5])
                m67 = jnp.maximum(g[6], g[7])
                m = jnp.maximum(jnp.maximum(m01, m23),
                                jnp.maximum(m45, m67))
                oa, ob = plsc.unpack(m, format=plsc.PackFormat.INTERLEAVED)
                out_v[_buf, 0, pl.ds(base, _L)] = oa
                out_v[_buf, 1, pl.ds(base, _L)] = ob
            c0 = cc * _CH
            pltpu.async_copy(out_v.at[buf, 0], out_hbm.at[ra, pl.ds(c0, _CH)],
                             sem_ot[buf])
            pltpu.async_copy(out_v.at[buf, 1], out_hbm.at[rb, pl.ds(c0, _CH)],
                             sem_ot[buf])

        _scope_gather.__exit__(None, None, None)

        for buf in range(2):
            pltpu.make_async_copy(out_v.at[buf, 0],
                                  out_hbm.at[ra, pl.ds(0, _CH)],
                                  sem_ot[buf]).wait()
            pltpu.make_async_copy(out_v.at[buf, 1],
                                  out_hbm.at[rb, pl.ds(0, _CH)],
                                  sem_ot[buf]).wait()
        return carry

    lax.fori_loop(0, _PAIRS_PW, pair_body, 0)


_mesh = plsc.VectorSubcoreMesh(core_axis_name="c", subcore_axis_name="s")

_pool = functools.partial(
    pl.kernel,
    mesh=_mesh,
    compiler_params=pltpu.CompilerParams(needs_layout_passes=False),
    out_type=jax.ShapeDtypeStruct((_ROWS, _N_OUT), jnp.float32),
    scratch_types=[
        pltpu.VMEM((_N_IN,), jnp.int32),
        pltpu.VMEM((2, 2, _SG), jnp.float32),
        pltpu.VMEM((2, _K, _CH), jnp.int32),
        pltpu.VMEM((2, 2, _CH), jnp.float32),
        pltpu.VMEM_SHARED((_K, _N_OUT), jnp.int32),
        pltpu.SemaphoreType.DMA,
        pltpu.SemaphoreType.DMA,
        pltpu.SemaphoreType.DMA,
        pltpu.SemaphoreType.DMA,
        pltpu.SemaphoreType.DMA,
        pltpu.SemaphoreType.DMA,
        pltpu.SemaphoreType.DMA,
    ],
)(_pool_body)


def kernel(x, neighbours):
    x2 = x.reshape(_ROWS, _N_IN)
    out2 = _pool(x2, neighbours)
    return out2.reshape(_B, _C, _N_OUT)


# speedup vs baseline: 1.0984x; 1.0984x over previous
import functools

import jax
import jax.numpy as jnp
from jax import lax
from jax.experimental import pallas as pl
from jax.experimental.pallas import tpu as pltpu
from jax.experimental.pallas import tpu_sc as plsc

_B, _C, _N_IN, _N_OUT, _K = 4, 128, 65536, 16384, 8
_NC, _NS, _L = 2, 16, 16
_NW = _NC * _NS
_ROWS = _B * _C
_PAIRS_PW = _ROWS // (2 * _NW)
_SG = 8192
_NSG = _N_IN // _SG
_CH = 1024
_NCHUNK = _N_OUT // _CH


def _pool_body(x_hbm, nbr_hbm, out_hbm, packed_v, stage_v, idx_v, out_v,
               idx_sh, sem_nbr, sem_st0, sem_st1, sem_ix0, sem_ix1,
               sem_ot0, sem_ot1):
    cid = lax.axis_index("c")
    sid = lax.axis_index("s")
    wid = sid * _NC + cid
    sem_st = (sem_st0, sem_st1)
    sem_ix = (sem_ix0, sem_ix1)
    sem_ot = (sem_ot0, sem_ot1)

    @pl.when(sid == 0)
    def _stage():
        pltpu.async_copy(nbr_hbm, idx_sh, sem_nbr).wait()

    plsc.subcore_barrier()

    def pair_body(p, carry):
        ra = (wid * _PAIRS_PW + p) * 2
        rb = ra + 1

        pltpu.async_copy(idx_sh.at[:, pl.ds(0, _CH)], idx_v.at[0], sem_ix[0])

        _scope_pack = jax.named_scope("pool_pack")
        _scope_pack.__enter__()
        pltpu.async_copy(x_hbm.at[ra, pl.ds(0, _SG)], stage_v.at[0, 0],
                         sem_st[0])
        pltpu.async_copy(x_hbm.at[rb, pl.ds(0, _SG)], stage_v.at[0, 1],
                         sem_st[0])
        for s in range(_NSG):
            buf = s % 2
            if s + 1 < _NSG:
                nbuf = 1 - buf
                off = (s + 1) * _SG
                pltpu.async_copy(x_hbm.at[ra, pl.ds(off, _SG)],
                                 stage_v.at[nbuf, 0], sem_st[nbuf])
                pltpu.async_copy(x_hbm.at[rb, pl.ds(off, _SG)],
                                 stage_v.at[nbuf, 1], sem_st[nbuf])
            pltpu.make_async_copy(x_hbm.at[ra, pl.ds(0, _SG)],
                                  stage_v.at[buf, 0], sem_st[buf]).wait()
            pltpu.make_async_copy(x_hbm.at[rb, pl.ds(0, _SG)],
                                  stage_v.at[buf, 1], sem_st[buf]).wait()

            @plsc.parallel_loop(0, _SG // _L, unroll=4)
            def pack_body(j, _buf=buf, _s=s):
                base = j * _L
                a = stage_v[_buf, 0, pl.ds(base, _L)]
                b = stage_v[_buf, 1, pl.ds(base, _L)]
                pw = plsc.bitcast(
                    plsc.pack(a, b, format=plsc.PackFormat.INTERLEAVED),
                    jnp.int32)
                packed_v[pl.ds(_s * _SG + base, _L)] = pw

        _scope_pack.__exit__(None, None, None)

        _scope_gather = jax.named_scope("pool_gather")
        _scope_gather.__enter__()
        for cc in range(_NCHUNK):
            buf = cc % 2
            if cc + 1 < _NCHUNK:
                nbuf = 1 - buf
                pltpu.async_copy(idx_sh.at[:, pl.ds((cc + 1) * _CH, _CH)],
                                 idx_v.at[nbuf], sem_ix[nbuf])
            pltpu.make_async_copy(idx_sh.at[:, pl.ds(0, _CH)],
                                  idx_v.at[buf], sem_ix[buf]).wait()
            if cc >= 2:
                pltpu.make_async_copy(out_v.at[buf, 0],
                                      out_hbm.at[ra, pl.ds(0, _CH)],
                                      sem_ot[buf]).wait()
                pltpu.make_async_copy(out_v.at[buf, 1],
                                      out_hbm.at[rb, pl.ds(0, _CH)],
                                      sem_ot[buf]).wait()

            @plsc.parallel_loop(0, _CH // _L, unroll=2)
            def vec_body(j, _buf=buf):
                base = j * _L
                g = []
                for kk in range(_K // 2):
                    w = idx_v[_buf, kk, pl.ds(base, _L)]
                    iv0 = jnp.bitwise_and(w, jnp.int32(0xFFFF))
                    iv1 = lax.shift_right_logical(w, 16)
                    for iv in (iv0, iv1):
                        gk = plsc.load_gather(packed_v, [iv])
                        g.append(plsc.bitcast(gk, jnp.bfloat16))
                m01 = jnp.maximum(g[0], g[1])
                m23 = jnp.maximum(g[2], g[3])
                m45 = jnp.maximum(g[4], g[5])
                m67 = jnp.maximum(g[6], g[7])
                m = jnp.maximum(jnp.maximum(m01, m23),
                                jnp.maximum(m45, m67))
                oa, ob = plsc.unpack(m, format=plsc.PackFormat.INTERLEAVED)
                out_v[_buf, 0, pl.ds(base, _L)] = oa
                out_v[_buf, 1, pl.ds(base, _L)] = ob
            c0 = cc * _CH
            pltpu.async_copy(out_v.at[buf, 0], out_hbm.at[ra, pl.ds(c0, _CH)],
                             sem_ot[buf])
            pltpu.async_copy(out_v.at[buf, 1], out_hbm.at[rb, pl.ds(c0, _CH)],
                             sem_ot[buf])

        _scope_gather.__exit__(None, None, None)

        for buf in range(2):
            pltpu.make_async_copy(out_v.at[buf, 0],
                                  out_hbm.at[ra, pl.ds(0, _CH)],
                                  sem_ot[buf]).wait()
            pltpu.make_async_copy(out_v.at[buf, 1],
                                  out_hbm.at[rb, pl.ds(0, _CH)],
                                  sem_ot[buf]).wait()
        return carry

    lax.fori_loop(0, _PAIRS_PW, pair_body, 0)


_mesh = plsc.VectorSubcoreMesh(core_axis_name="c", subcore_axis_name="s")

_pool = functools.partial(
    pl.kernel,
    mesh=_mesh,
    compiler_params=pltpu.CompilerParams(needs_layout_passes=False),
    out_type=jax.ShapeDtypeStruct((_ROWS, _N_OUT), jnp.float32),
    scratch_types=[
        pltpu.VMEM((_N_IN,), jnp.int32),
        pltpu.VMEM((2, 2, _SG), jnp.float32),
        pltpu.VMEM((2, _K // 2, _CH), jnp.int32),
        pltpu.VMEM((2, 2, _CH), jnp.float32),
        pltpu.VMEM_SHARED((_K // 2, _N_OUT), jnp.int32),
        pltpu.SemaphoreType.DMA,
        pltpu.SemaphoreType.DMA,
        pltpu.SemaphoreType.DMA,
        pltpu.SemaphoreType.DMA,
        pltpu.SemaphoreType.DMA,
        pltpu.SemaphoreType.DMA,
        pltpu.SemaphoreType.DMA,
    ],
)(_pool_body)


def kernel(x, neighbours):
    x2 = x.reshape(_ROWS, _N_IN)
    nbr_packed = jnp.bitwise_or(
        jnp.left_shift(neighbours[1::2, :], 16), neighbours[0::2, :])
    out2 = _pool(x2, nbr_packed)
    return out2.reshape(_B, _C, _N_OUT)


# speedup vs baseline: 1.3288x; 1.2098x over previous
import functools

import jax
import jax.numpy as jnp
from jax import lax
from jax.experimental import pallas as pl
from jax.experimental.pallas import tpu as pltpu
from jax.experimental.pallas import tpu_sc as plsc

_B, _C, _N_IN, _N_OUT, _K = 4, 128, 65536, 16384, 8
_NC, _NS, _L = 2, 16, 16
_NW = _NC * _NS
_ROWS = _B * _C
_PAIRS_PW = _ROWS // (2 * _NW)
_SG = 8192
_NSG = _N_IN // _SG
_CH = 1024
_NCHUNK = _N_OUT // _CH


def _pool_body(x_hbm, nbr_hbm, out_hbm, packed_v, stage_v, idx_v, out_v,
               idx_sh, sem_nbr, sem_st0, sem_st1, sem_ix0, sem_ix1,
               sem_ot0, sem_ot1):
    cid = lax.axis_index("c")
    sid = lax.axis_index("s")
    wid = sid * _NC + cid
    sem_st = (sem_st0, sem_st1)
    sem_ix = (sem_ix0, sem_ix1)
    sem_ot = (sem_ot0, sem_ot1)

    @pl.when(sid == 0)
    def _stage():
        pltpu.async_copy(nbr_hbm, idx_sh, sem_nbr).wait()

    plsc.subcore_barrier()

    ra0 = wid * _PAIRS_PW * 2
    pltpu.async_copy(x_hbm.at[pl.ds(ra0, 2), pl.ds(0, _SG)], stage_v.at[0],
                     sem_st[0])

    def pair_body(p, carry):
        ra = (wid * _PAIRS_PW + p) * 2
        rb = ra + 1
        ra_nx = jnp.minimum(ra + 2, _ROWS - 2)

        pltpu.async_copy(idx_sh.at[:, pl.ds(0, _CH)], idx_v.at[0], sem_ix[0])

        _scope_pack = jax.named_scope("pool_pack")
        _scope_pack.__enter__()
        for s in range(_NSG):
            buf = s % 2
            nbuf = 1 - buf
            if s + 1 < _NSG:
                off = (s + 1) * _SG
                pltpu.async_copy(x_hbm.at[pl.ds(ra, 2), pl.ds(off, _SG)],
                                 stage_v.at[nbuf], sem_st[nbuf])
            else:
                pltpu.async_copy(x_hbm.at[pl.ds(ra_nx, 2), pl.ds(0, _SG)],
                                 stage_v.at[nbuf], sem_st[nbuf])
            pltpu.make_async_copy(x_hbm.at[pl.ds(ra, 2), pl.ds(0, _SG)],
                                  stage_v.at[buf], sem_st[buf]).wait()

            @plsc.parallel_loop(0, _SG // _L, unroll=8)
            def pack_body(j, _buf=buf, _s=s):
                base = j * _L
                a = stage_v[_buf, 0, pl.ds(base, _L)]
                b = stage_v[_buf, 1, pl.ds(base, _L)]
                pw = plsc.bitcast(
                    plsc.pack(a, b, format=plsc.PackFormat.INTERLEAVED),
                    jnp.int32)
                packed_v[pl.ds(_s * _SG + base, _L)] = pw

        _scope_pack.__exit__(None, None, None)

        _scope_gather = jax.named_scope("pool_gather")
        _scope_gather.__enter__()
        for cc in range(_NCHUNK):
            buf = cc % 2
            if cc + 1 < _NCHUNK:
                nbuf = 1 - buf
                pltpu.async_copy(idx_sh.at[:, pl.ds((cc + 1) * _CH, _CH)],
                                 idx_v.at[nbuf], sem_ix[nbuf])
            pltpu.make_async_copy(idx_sh.at[:, pl.ds(0, _CH)],
                                  idx_v.at[buf], sem_ix[buf]).wait()
            if cc >= 2:
                pltpu.make_async_copy(out_v.at[buf, 0],
                                      out_hbm.at[ra, pl.ds(0, _CH)],
                                      sem_ot[buf]).wait()
                pltpu.make_async_copy(out_v.at[buf, 1],
                                      out_hbm.at[rb, pl.ds(0, _CH)],
                                      sem_ot[buf]).wait()

            @plsc.parallel_loop(0, _CH // _L, unroll=4)
            def vec_body(j, _buf=buf):
                base = j * _L
                g = []
                for kk in range(_K // 2):
                    w = idx_v[_buf, kk, pl.ds(base, _L)]
                    iv0 = jnp.bitwise_and(w, jnp.int32(0xFFFF))
                    iv1 = lax.shift_right_logical(w, 16)
                    for iv in (iv0, iv1):
                        gk = plsc.load_gather(packed_v, [iv])
                        g.append(plsc.bitcast(gk, jnp.bfloat16))
                m01 = jnp.maximum(g[0], g[1])
                m23 = jnp.maximum(g[2], g[3])
                m45 = jnp.maximum(g[4], g[5])
                m67 = jnp.maximum(g[6], g[7])
                m = jnp.maximum(jnp.maximum(m01, m23),
                                jnp.maximum(m45, m67))
                oa, ob = plsc.unpack(m, format=plsc.PackFormat.INTERLEAVED)
                out_v[_buf, 0, pl.ds(base, _L)] = oa
                out_v[_buf, 1, pl.ds(base, _L)] = ob
            c0 = cc * _CH
            pltpu.async_copy(out_v.at[buf, 0], out_hbm.at[ra, pl.ds(c0, _CH)],
                             sem_ot[buf])
            pltpu.async_copy(out_v.at[buf, 1], out_hbm.at[rb, pl.ds(c0, _CH)],
                             sem_ot[buf])

        _scope_gather.__exit__(None, None, None)

        for buf in range(2):
            pltpu.make_async_copy(out_v.at[buf, 0],
                                  out_hbm.at[ra, pl.ds(0, _CH)],
                                  sem_ot[buf]).wait()
            pltpu.make_async_copy(out_v.at[buf, 1],
                                  out_hbm.at[rb, pl.ds(0, _CH)],
                                  sem_ot[buf]).wait()
        return carry

    lax.fori_loop(0, _PAIRS_PW, pair_body, 0)


_mesh = plsc.VectorSubcoreMesh(core_axis_name="c", subcore_axis_name="s")

_pool = functools.partial(
    pl.kernel,
    mesh=_mesh,
    compiler_params=pltpu.CompilerParams(needs_layout_passes=False),
    out_type=jax.ShapeDtypeStruct((_ROWS, _N_OUT), jnp.float32),
    scratch_types=[
        pltpu.VMEM((_N_IN,), jnp.int32),
        pltpu.VMEM((2, 2, _SG), jnp.float32),
        pltpu.VMEM((2, _K // 2, _CH), jnp.int32),
        pltpu.VMEM((2, 2, _CH), jnp.float32),
        pltpu.VMEM_SHARED((_K // 2, _N_OUT), jnp.int32),
        pltpu.SemaphoreType.DMA,
        pltpu.SemaphoreType.DMA,
        pltpu.SemaphoreType.DMA,
        pltpu.SemaphoreType.DMA,
        pltpu.SemaphoreType.DMA,
        pltpu.SemaphoreType.DMA,
        pltpu.SemaphoreType.DMA,
    ],
)(_pool_body)


def kernel(x, neighbours):
    x2 = x.reshape(_ROWS, _N_IN)
    nbr_packed = jnp.bitwise_or(
        jnp.left_shift(neighbours[1::2, :], 16), neighbours[0::2, :])
    out2 = _pool(x2, nbr_packed)
    return out2.reshape(_B, _C, _N_OUT)
